# speedup vs baseline: 2.6072x; 2.6072x over previous
import functools

import jax
import jax.numpy as jnp
from jax import lax
from jax.experimental import pallas as pl
from jax.experimental.pallas import tpu as pltpu
from jax.experimental.pallas import tpu_sc as plsc

EMBED_DIM = 128
CHUNK = 128


@functools.lru_cache(maxsize=None)
def _emb_lookup(B, V, D):
    info = plsc.get_sparse_core_info()
    NC, NS = info.num_cores, info.num_subcores
    NW = NC * NS
    assert B % (NW * CHUNK) == 0
    b_per_w = B // NW
    nchunks = b_per_w // CHUNK
    mesh = plsc.VectorSubcoreMesh(core_axis_name="c", subcore_axis_name="s")

    @functools.partial(
        pl.kernel,
        mesh=mesh,
        out_type=jax.ShapeDtypeStruct((B, D), jnp.float32),
        scratch_types=[
            pltpu.VMEM((CHUNK,), jnp.int32),
            pltpu.VMEM((CHUNK, D), jnp.float32),
            pltpu.SemaphoreType.DMA,
        ],
    )
    def k(x_hbm, table_hbm, out_hbm, idx_v, rows_v, sem):
        wid = lax.axis_index("s") * NC + lax.axis_index("c")
        base = wid * b_per_w

        def body(g, carry):
            off = base + g * CHUNK
            pltpu.sync_copy(x_hbm.at[pl.ds(off, CHUNK)], idx_v)
            pltpu.async_copy(table_hbm.at[idx_v], rows_v, sem).wait()
            pltpu.sync_copy(rows_v, out_hbm.at[pl.ds(off, CHUNK)])
            return carry

        lax.fori_loop(0, nchunks, body, 0)

    return k


def kernel(x, table):
    B = x.shape[0] * x.shape[1]
    V, D = table.shape
    out = _emb_lookup(B, V, D)(x.reshape(B), table)
    return out.reshape(x.shape[0], x.shape[1], D)


# speedup vs baseline: 2.7875x; 1.0691x over previous
import functools

import jax
import jax.numpy as jnp
from jax import lax
from jax.experimental import pallas as pl
from jax.experimental.pallas import tpu as pltpu
from jax.experimental.pallas import tpu_sc as plsc

EMBED_DIM = 128
CHUNK = 128


@functools.lru_cache(maxsize=None)
def _emb_lookup(B, V, D):
    info = plsc.get_sparse_core_info()
    NC, NS = info.num_cores, info.num_subcores
    NW = NC * NS
    assert B % (NW * CHUNK) == 0
    b_per_w = B // NW
    nchunks = b_per_w // CHUNK
    assert nchunks % 2 == 0
    mesh = plsc.VectorSubcoreMesh(core_axis_name="c", subcore_axis_name="s")

    @functools.partial(
        pl.kernel,
        mesh=mesh,
        out_type=jax.ShapeDtypeStruct((B, D), jnp.float32),
        scratch_types=[
            pltpu.VMEM((nchunks, CHUNK), jnp.int32),
            pltpu.VMEM((2, CHUNK, D), jnp.float32),
            pltpu.SemaphoreType.DMA,
            pltpu.SemaphoreType.DMA,
        ],
    )
    def k(x_hbm, table_hbm, out_hbm, idx_v, rows_v, gsem0, gsem1):
        wid = lax.axis_index("s") * NC + lax.axis_index("c")
        base = wid * b_per_w
        pltpu.sync_copy(x_hbm.at[wid], idx_v)

        pltpu.async_copy(table_hbm.at[idx_v.at[0]], rows_v.at[0], gsem0)

        def body(i, carry):
            g0 = 2 * i
            pltpu.async_copy(table_hbm.at[idx_v.at[g0 + 1]], rows_v.at[1], gsem1)
            pltpu.make_async_copy(
                table_hbm.at[idx_v.at[g0]], rows_v.at[0], gsem0
            ).wait()
            pltpu.sync_copy(
                rows_v.at[0], out_hbm.at[pl.ds(base + g0 * CHUNK, CHUNK)]
            )

            @pl.when(g0 + 2 < nchunks)
            def _():
                pltpu.async_copy(
                    table_hbm.at[idx_v.at[g0 + 2]], rows_v.at[0], gsem0
                )

            pltpu.make_async_copy(
                table_hbm.at[idx_v.at[g0 + 1]], rows_v.at[1], gsem1
            ).wait()
            pltpu.sync_copy(
                rows_v.at[1], out_hbm.at[pl.ds(base + (g0 + 1) * CHUNK, CHUNK)]
            )
            return carry

        lax.fori_loop(0, nchunks // 2, body, 0)

    return k


def kernel(x, table):
    B = x.shape[0] * x.shape[1]
    V, D = table.shape
    info = plsc.get_sparse_core_info()
    NW = info.num_cores * info.num_subcores
    xr = x.reshape(NW, B // (NW * CHUNK), CHUNK)
    out = _emb_lookup(B, V, D)(xr, table)
    return out.reshape(x.shape[0], x.shape[1], D)


# speedup vs baseline: 4.3943x; 1.5765x over previous
import functools

import jax
import jax.numpy as jnp
from jax import lax
from jax.experimental import pallas as pl
from jax.experimental.pallas import tpu as pltpu
from jax.experimental.pallas import tpu_sc as plsc


@functools.lru_cache(maxsize=None)
def _emb_lookup(NB, H, V, D):
    info = plsc.get_sparse_core_info()
    NC, NS = info.num_cores, info.num_subcores
    NW = NC * NS
    assert NB % NW == 0
    epw = NB // NW
    assert epw % 2 == 0
    mesh = plsc.VectorSubcoreMesh(core_axis_name="c", subcore_axis_name="s")

    @functools.partial(
        pl.kernel,
        mesh=mesh,
        out_type=jax.ShapeDtypeStruct((NB, H, D), jnp.float32),
        scratch_types=[
            pltpu.VMEM((epw, H), jnp.int32),
            pltpu.VMEM((2, H, D), jnp.float32),
            pltpu.SemaphoreType.DMA,
            pltpu.SemaphoreType.DMA,
        ],
        compiler_params=pltpu.CompilerParams(use_tc_tiling_on_sc=True),
    )
    def k(x_hbm, table_hbm, out_hbm, idx_v, rows_v, gsem0, gsem1):
        wid = lax.axis_index("s") * NC + lax.axis_index("c")
        base = wid * epw
        pltpu.sync_copy(x_hbm.at[pl.ds(base, epw)], idx_v)

        pltpu.async_copy(table_hbm.at[idx_v.at[0]], rows_v.at[0], gsem0)

        def body(i, carry):
            e0 = 2 * i
            pltpu.async_copy(table_hbm.at[idx_v.at[e0 + 1]], rows_v.at[1], gsem1)
            pltpu.make_async_copy(
                table_hbm.at[idx_v.at[e0]], rows_v.at[0], gsem0
            ).wait()
            pltpu.sync_copy(rows_v.at[0], out_hbm.at[base + e0])

            @pl.when(e0 + 2 < epw)
            def _():
                pltpu.async_copy(
                    table_hbm.at[idx_v.at[e0 + 2]], rows_v.at[0], gsem0
                )

            pltpu.make_async_copy(
                table_hbm.at[idx_v.at[e0 + 1]], rows_v.at[1], gsem1
            ).wait()
            pltpu.sync_copy(rows_v.at[1], out_hbm.at[base + e0 + 1])
            return carry

        lax.fori_loop(0, epw // 2, body, 0)

    return k


def kernel(x, table):
    NB, H = x.shape
    V, D = table.shape
    return _emb_lookup(NB, H, V, D)(x, table)


# speedup vs baseline: 6.5560x; 1.4919x over previous
import functools

import jax
import jax.numpy as jnp
from jax import lax
from jax.experimental import pallas as pl
from jax.experimental.pallas import tpu as pltpu
from jax.experimental.pallas import tpu_sc as plsc

CHUNK = 128


@functools.lru_cache(maxsize=None)
def _emb_lookup(B, V, D):
    info = plsc.get_sparse_core_info()
    NC, NS = info.num_cores, info.num_subcores
    NW = NC * NS
    assert B % (NW * CHUNK) == 0
    b_per_w = B // NW
    nchunks = b_per_w // CHUNK
    assert nchunks % 2 == 0
    mesh = plsc.VectorSubcoreMesh(core_axis_name="c", subcore_axis_name="s")

    @functools.partial(
        pl.kernel,
        mesh=mesh,
        out_type=jax.ShapeDtypeStruct((B, D), jnp.float32),
        scratch_types=[
            pltpu.VMEM((nchunks, CHUNK), jnp.int32),
            pltpu.VMEM((2, CHUNK, D), jnp.float32),
            pltpu.SemaphoreType.DMA,
            pltpu.SemaphoreType.DMA,
        ],
    )
    def k(x_hbm, table_hbm, out_hbm, idx_v, rows_v, gsem0, gsem1):
        wid = lax.axis_index("s") * NC + lax.axis_index("c")
        base = wid * b_per_w
        pltpu.sync_copy(x_hbm.at[wid], idx_v)

        pltpu.async_copy(table_hbm.at[idx_v.at[0]], rows_v.at[0], gsem0)

        def body(i, carry):
            g0 = 2 * i
            pltpu.async_copy(table_hbm.at[idx_v.at[g0 + 1]], rows_v.at[1], gsem1)
            pltpu.make_async_copy(
                table_hbm.at[idx_v.at[g0]], rows_v.at[0], gsem0
            ).wait()
            pltpu.sync_copy(
                rows_v.at[0], out_hbm.at[pl.ds(base + g0 * CHUNK, CHUNK)]
            )

            @pl.when(g0 + 2 < nchunks)
            def _():
                pltpu.async_copy(
                    table_hbm.at[idx_v.at[g0 + 2]], rows_v.at[0], gsem0
                )

            pltpu.make_async_copy(
                table_hbm.at[idx_v.at[g0 + 1]], rows_v.at[1], gsem1
            ).wait()
            pltpu.sync_copy(
                rows_v.at[1], out_hbm.at[pl.ds(base + (g0 + 1) * CHUNK, CHUNK)]
            )
            return carry

        lax.fori_loop(0, nchunks // 2, body, 0)

    return k


def kernel(x, table):
    NB, H = x.shape
    V, D = table.shape
    B = NB * H
    info = plsc.get_sparse_core_info()
    NW = info.num_cores * info.num_subcores
    xr = x.T.reshape(NW, B // (NW * CHUNK), CHUNK)
    out = _emb_lookup(B, V, D)(xr, table)
    return out.reshape(H, NB, D).transpose(1, 0, 2)


# speedup vs baseline: 15.5136x; 2.3663x over previous
import functools

import jax
import jax.numpy as jnp
from jax import lax
from jax.experimental import pallas as pl
from jax.experimental.pallas import tpu as pltpu
from jax.experimental.pallas import tpu_sc as plsc

CHUNK = 128


@functools.lru_cache(maxsize=None)
def _emb_lookup(B, V, D):
    info = plsc.get_sparse_core_info()
    NC, NS = info.num_cores, info.num_subcores
    NW = NC * NS
    assert B % (NW * CHUNK) == 0
    b_per_w = B // NW
    nchunks = b_per_w // CHUNK
    assert nchunks % 2 == 0
    mesh = plsc.VectorSubcoreMesh(core_axis_name="c", subcore_axis_name="s")

    @functools.partial(
        pl.kernel,
        mesh=mesh,
        out_type=jax.ShapeDtypeStruct((B, D), jnp.float32),
        scratch_types=[
            pltpu.VMEM((nchunks, CHUNK), jnp.int32),
            pltpu.VMEM((2, CHUNK, D), jnp.float32),
            pltpu.VMEM_SHARED((V, D), jnp.float32),
            pltpu.SemaphoreType.DMA,
            pltpu.SemaphoreType.DMA,
        ],
    )
    def k(x_hbm, table_hbm, out_hbm, idx_v, rows_v, table_sp, gsem0, gsem1):
        sid = lax.axis_index("s")
        wid = sid * NC + lax.axis_index("c")
        base = wid * b_per_w

        @pl.when(sid == 0)
        def _():
            pltpu.sync_copy(table_hbm, table_sp)

        pltpu.sync_copy(x_hbm.at[wid], idx_v)
        plsc.subcore_barrier()

        pltpu.async_copy(table_sp.at[idx_v.at[0]], rows_v.at[0], gsem0)

        def body(i, carry):
            g0 = 2 * i
            pltpu.async_copy(table_sp.at[idx_v.at[g0 + 1]], rows_v.at[1], gsem1)
            pltpu.make_async_copy(
                table_sp.at[idx_v.at[g0]], rows_v.at[0], gsem0
            ).wait()
            pltpu.sync_copy(
                rows_v.at[0], out_hbm.at[pl.ds(base + g0 * CHUNK, CHUNK)]
            )

            @pl.when(g0 + 2 < nchunks)
            def _():
                pltpu.async_copy(
                    table_sp.at[idx_v.at[g0 + 2]], rows_v.at[0], gsem0
                )

            pltpu.make_async_copy(
                table_sp.at[idx_v.at[g0 + 1]], rows_v.at[1], gsem1
            ).wait()
            pltpu.sync_copy(
                rows_v.at[1], out_hbm.at[pl.ds(base + (g0 + 1) * CHUNK, CHUNK)]
            )
            return carry

        lax.fori_loop(0, nchunks // 2, body, 0)

    return k


def kernel(x, table):
    NB, H = x.shape
    V, D = table.shape
    B = NB * H
    info = plsc.get_sparse_core_info()
    NW = info.num_cores * info.num_subcores
    xr = x.T.reshape(NW, B // (NW * CHUNK), CHUNK)
    out = _emb_lookup(B, V, D)(xr, table)
    return out.reshape(H, NB, D).transpose(1, 0, 2)
